# baseline (speedup 1.0000x reference)
import jax
import jax.numpy as jnp
from jax import lax
from jax.experimental import pallas as pl
from jax.experimental.pallas import tpu as pltpu

_N = 10000
_E = 160000
_F = 128
_F3 = 384


def _sigmoid(v):
    return 0.5 * (jnp.tanh(0.5 * v) + 1.0)


def _body(pe2_ref, nperm_ref, ndeg_ref, e2s_ref, niter_ref,
          x_ref, WiouT_ref, biou_ref, UiouT_ref, buiou_ref,
          WfT_ref, bfsum_ref, UfT_ref,
          h_ref,
          xfb_ref, c_ref, hUf_ref, obuf_ref, fcbuf_ref):
    h_ref[...] = jnp.zeros((_N, _F), jnp.float32)
    c_ref[...] = jnp.zeros((_N, _F), jnp.float32)
    hUf_ref[...] = jnp.zeros((_N, _F), jnp.float32)
    xfb_ref[...] = (
        jnp.dot(x_ref[...], WfT_ref[...], preferred_element_type=jnp.float32)
        + bfsum_ref[...]
    )

    def iter_body(r, nptr):
        def a_body(q, _):
            v = nperm_ref[q]
            pk = pe2_ref[e2s_ref[q] + (r - 1)]
            ch = pk & 16383
            f = _sigmoid(xfb_ref[pl.ds(v, 1), :] + hUf_ref[pl.ds(ch, 1), :])
            fcbuf_ref[pl.ds(q - nptr, 1), :] = f * c_ref[pl.ds(ch, 1), :]
            return 0

        lax.cond(r > 0,
                 lambda: lax.fori_loop(nptr, _N, a_body, 0),
                 lambda: 0)

        def b_cond(q):
            return (q < _N) & (ndeg_ref[jnp.minimum(q, _N - 1)] == r)

        def b_body(q):
            v = nperm_ref[q]
            e2 = e2s_ref[q]

            def acc_body(j, acc):
                pk = pe2_ref[e2 + j]
                m = (pk >> 14).astype(jnp.float32)
                ch = pk & 16383
                return acc + m * h_ref[pl.ds(ch, 1), :]

            hs = lax.fori_loop(0, r, acc_body,
                               jnp.zeros((1, _F), jnp.float32))
            iou = (
                jnp.dot(x_ref[pl.ds(v, 1), :], WiouT_ref[...],
                        preferred_element_type=jnp.float32)
                + biou_ref[...]
            )
            term = (
                jnp.dot(hs, UiouT_ref[...],
                        preferred_element_type=jnp.float32)
                + buiou_ref[...]
            )
            iou = iou + jnp.where(r > 0, 1.0, 0.0) * term
            gi = _sigmoid(iou[:, :_F])
            go = _sigmoid(iou[:, _F:2 * _F])
            gu = jnp.tanh(iou[:, 2 * _F:])
            c_ref[pl.ds(v, 1), :] = gi * gu
            obuf_ref[pl.ds(v, 1), :] = go
            return q + 1

        nend = lax.while_loop(b_cond, b_body, nptr)

        def c_body(q, _):
            v = nperm_ref[q]
            c_ref[pl.ds(v, 1), :] = (
                c_ref[pl.ds(v, 1), :] + fcbuf_ref[pl.ds(q - nptr, 1), :]
            )
            return 0

        lax.cond(r > 0,
                 lambda: lax.fori_loop(nptr, _N, c_body, 0),
                 lambda: 0)

        def d_body(q, _):
            v = nperm_ref[q]
            hrow = obuf_ref[pl.ds(v, 1), :] * jnp.tanh(c_ref[pl.ds(v, 1), :])
            h_ref[pl.ds(v, 1), :] = hrow
            hUf_ref[pl.ds(v, 1), :] = jnp.dot(
                hrow, UfT_ref[...], preferred_element_type=jnp.float32)
            return 0

        lax.fori_loop(nptr, nend, d_body, 0)
        return nend

    lax.fori_loop(0, niter_ref[0], iter_body, 0)


def kernel(x, edge_index, edge_feats, edge_types,
           W_iou_w, W_iou_b, U_iou_w, U_iou_b,
           W_f_w, W_f_b, U_f_w, U_f_b):
    del edge_feats, edge_types

    parents = edge_index[0].astype(jnp.int32)
    children = edge_index[1].astype(jnp.int32)
    order = jnp.argsort(parents, stable=True)
    sp = parents[order]
    idx = jnp.arange(_E, dtype=jnp.int32)
    is_grp = jnp.concatenate([jnp.zeros((1,), dtype=bool), sp[1:] != sp[:-1]])
    group_start = jnp.where(is_grp, idx, 0)
    group_start = lax.cummax(group_start, axis=0)
    pos = idx - group_start + 1
    rank = jnp.zeros(_E, dtype=jnp.int32).at[order].set(pos)
    deg = jnp.bincount(parents, length=_N).astype(jnp.int32)
    niter = (deg.max() + 1).astype(jnp.int32).reshape(1)

    nperm = jnp.argsort(deg, stable=True).astype(jnp.int32)
    ndeg = deg[nperm]
    e2start = jnp.concatenate(
        [jnp.zeros((1,), jnp.int32), jnp.cumsum(ndeg)[:-1].astype(jnp.int32)])

    inv_nperm = jnp.zeros(_N, dtype=jnp.int32).at[nperm].set(
        jnp.arange(_N, dtype=jnp.int32))
    wperm = jnp.argsort(inv_nperm[parents], stable=True)
    mask2 = (deg[children] < rank).astype(jnp.int32)
    packed2 = (mask2 * (1 << 14) + children)[wperm]

    smem = pl.BlockSpec(memory_space=pltpu.SMEM)
    vmem = pl.BlockSpec(memory_space=pltpu.VMEM)
    out = pl.pallas_call(
        _body,
        out_shape=jax.ShapeDtypeStruct((_N, _F), jnp.float32),
        in_specs=[smem] * 5 + [vmem] * 8,
        out_specs=vmem,
        scratch_shapes=[
            pltpu.VMEM((_N, _F), jnp.float32),
            pltpu.VMEM((_N, _F), jnp.float32),
            pltpu.VMEM((_N, _F), jnp.float32),
            pltpu.VMEM((_N, _F), jnp.float32),
            pltpu.VMEM((_N, _F), jnp.float32),
        ],
    )(
        packed2, nperm, ndeg, e2start, niter,
        x,
        W_iou_w.T, W_iou_b.reshape(1, _F3),
        U_iou_w.T, U_iou_b.reshape(1, _F3),
        W_f_w.T, (W_f_b + U_f_b).reshape(1, _F),
        U_f_w.T,
    )
    return out
